# baseline (speedup 1.0000x reference)
import functools

import jax
import jax.numpy as jnp
from jax import lax
from jax.experimental import pallas as pl
from jax.experimental.pallas import tpu as pltpu
from jax.experimental.pallas import tpu_sc as plsc

_N = 2048
_C = 2048
_H = 16
_D = 128
_U = 40
_SCALE = _D ** -0.5


def _qkv_kernel(x_ref, w_ref, b_ref, out_ref):
    acc = jax.lax.dot_general(
        x_ref[...], w_ref[...], (((1,), (0,)), ((), ())),
        preferred_element_type=jnp.float32)
    acc = acc + b_ref[0].astype(jnp.float32)
    for j in range(4):
        out_ref[0, j, :, :] = acc[:, j * _D:(j + 1) * _D]


def _qkv_proj(x_bf, w_bf, b):
    return pl.pallas_call(
        _qkv_kernel,
        grid=(3, 4),
        in_specs=[
            pl.BlockSpec((_N, _C), lambda s, hh: (0, 0)),
            pl.BlockSpec((_C, 512), lambda s, hh: (0, s * 4 + hh)),
            pl.BlockSpec((1, 1, 512), lambda s, hh: (s * 4 + hh, 0, 0)),
        ],
        out_specs=pl.BlockSpec((1, 4, _N, _D), lambda s, hh: (s, hh, 0, 0)),
        out_shape=jax.ShapeDtypeStruct((3, _H, _N, _D), jnp.float32),
    )(x_bf, w_bf, b)


def _topk_kernel(q_ref, idx_ref):
    q = q_ref[0]
    norms2 = jnp.sum(q * q, axis=-1)
    iota_n = jax.lax.broadcasted_iota(jnp.int32, (_H, _N), 1)
    vals = norms2
    picks = []
    for _ in range(_U):
        m = jnp.max(vals, axis=1, keepdims=True)
        cand = jnp.where(vals == m, iota_n, _N)
        sel = jnp.min(cand, axis=1, keepdims=True)
        picks.append(sel)
        vals = jnp.where(iota_n == sel, -jnp.inf, vals)
    idx = jnp.concatenate(picks, axis=1)
    head = jax.lax.broadcasted_iota(jnp.int32, (_H, _U), 0)
    idx_ref[:, 0, :] = idx + head * _N


def _topk(qkv):
    return pl.pallas_call(
        _topk_kernel,
        grid=(1,),
        in_specs=[pl.BlockSpec((1, _H, _N, _D), lambda i: (0, 0, 0, 0))],
        out_specs=pl.BlockSpec((_H, 1, _U), lambda i: (0, 0, 0)),
        out_shape=jax.ShapeDtypeStruct((_H, 1, _U), jnp.int32),
    )(qkv)


def _sc_gather(qkv_rows, idx_flat):
    mesh = plsc.VectorSubcoreMesh(core_axis_name="c", subcore_axis_name="s")

    @functools.partial(
        pl.kernel, mesh=mesh,
        out_type=jax.ShapeDtypeStruct((_H * _U, _D), jnp.float32),
        scratch_types=[
            pltpu.VMEM((_U,), jnp.int32),
            pltpu.VMEM((_U, _D), jnp.float32),
            pltpu.SemaphoreType.DMA,
        ],
    )
    def gather(table_hbm, idx_hbm, out_hbm, idx_v, rows_v, sem):
        wid = lax.axis_index("s") * 2 + lax.axis_index("c")

        @pl.when(wid < _H)
        def _():
            base = wid * _U
            pltpu.sync_copy(idx_hbm.at[pl.ds(base, _U)], idx_v)
            pltpu.async_copy(table_hbm.at[idx_v], rows_v, sem).wait()
            pltpu.sync_copy(rows_v, out_hbm.at[pl.ds(base, _U)])

    return gather(qkv_rows, idx_flat)


def _attn_kernel(qred_ref, k_ref, v_ref, idx_ref, out_ref):
    h = pl.program_id(0)
    q_red = qred_ref[0]
    k = k_ref[0, 0]
    v = v_ref[0, 0]
    local_idx = idx_ref[0, 0:1, :] - h * _N
    iota_n = jax.lax.broadcasted_iota(jnp.int32, (_N, _U), 0)
    onehot_t = (iota_n == local_idx).astype(jnp.float32)

    s = jax.lax.dot_general(
        q_red, k, (((1,), (1,)), ((), ())),
        preferred_element_type=jnp.float32) * _SCALE

    p = s - jnp.max(s, axis=1, keepdims=True)
    p = jnp.exp(p)
    p = p / jnp.sum(p, axis=1, keepdims=True)
    p2 = p - jnp.max(p, axis=1, keepdims=True)
    p2 = jnp.exp(p2)
    p2 = p2 / jnp.sum(p2, axis=1, keepdims=True)

    rows = jax.lax.dot_general(
        p2, v, (((1,), (0,)), ((), ())), preferred_element_type=jnp.float32)
    scat = jax.lax.dot_general(
        onehot_t, rows, (((1,), (0,)), ((), ())), preferred_element_type=jnp.float32)
    selected = jnp.sum(onehot_t, axis=1, keepdims=True) > 0.0
    out = jnp.where(selected, scat, jnp.nan)
    out_ref[...] = out.astype(jnp.bfloat16)


def _sparse_attn(q_red, qkv, idx_flat):
    return pl.pallas_call(
        _attn_kernel,
        grid=(_H,),
        in_specs=[
            pl.BlockSpec((1, _U, _D), lambda h: (h, 0, 0)),
            pl.BlockSpec((1, 1, _N, _D), lambda h: (1, h, 0, 0)),
            pl.BlockSpec((1, 1, _N, _D), lambda h: (2, h, 0, 0)),
            pl.BlockSpec((1, 1, _U), lambda h: (h, 0, 0)),
        ],
        out_specs=pl.BlockSpec((_N, _D), lambda h: (0, h)),
        out_shape=jax.ShapeDtypeStruct((_N, _C), jnp.bfloat16),
    )(q_red, qkv, qkv, idx_flat)


def _fc_kernel(x_ref, w_ref, b_ref, out_ref):
    acc = jax.lax.dot_general(
        x_ref[...], w_ref[...], (((1,), (0,)), ((), ())),
        preferred_element_type=jnp.float32)
    out_ref[...] = acc + b_ref[...]


def _fc(attn_out, w_bf, b):
    return pl.pallas_call(
        _fc_kernel,
        grid=(4,),
        in_specs=[
            pl.BlockSpec((512, _C), lambda i: (i, 0)),
            pl.BlockSpec((_C, _C), lambda i: (0, 0)),
            pl.BlockSpec((1, _C), lambda i: (0, 0)),
        ],
        out_specs=pl.BlockSpec((512, _C), lambda i: (i, 0)),
        out_shape=jax.ShapeDtypeStruct((_N, _C), jnp.float32),
    )(attn_out, w_bf, b)


@jax.jit
def kernel(query, W_qkv, b_qkv, W_fc, b_fc):
    B, N, C = query.shape
    x_bf = query.reshape(N, C).astype(jnp.bfloat16)
    qkv = _qkv_proj(x_bf, W_qkv.astype(jnp.bfloat16), b_qkv.reshape(12, 1, 512))
    idx_flat = _topk(qkv)
    q_red = _sc_gather(qkv.reshape(3 * _H * _N, _D), idx_flat.reshape(_H * _U))
    attn_out = _sparse_attn(q_red.reshape(_H, _U, _D), qkv, idx_flat)
    out = _fc(attn_out, W_fc.astype(jnp.bfloat16), b_fc.reshape(1, C))
    return out.reshape(B, N, C)


# speedup vs baseline: 1.2796x; 1.2796x over previous
import functools

import jax
import jax.numpy as jnp
from jax import lax
from jax.experimental import pallas as pl
from jax.experimental.pallas import tpu as pltpu
from jax.experimental.pallas import tpu_sc as plsc

_N = 2048
_C = 2048
_H = 16
_D = 128
_U = 40
_SCALE = _D ** -0.5


def _qkv_kernel(x_ref, w_ref, b_ref, q_ref, kv_ref, n2_ref, xbf_ref):
    hh = pl.program_id(0)
    s = pl.program_id(1)

    @pl.when((hh == 0) & (s == 0))
    def _():
        xbf_ref[...] = x_ref[...].astype(jnp.bfloat16)

    acc = jax.lax.dot_general(
        xbf_ref[...], w_ref[...].astype(jnp.bfloat16), (((1,), (0,)), ((), ())),
        preferred_element_type=jnp.float32)
    acc = acc + b_ref[0]

    @pl.when(s == 0)
    def _():
        for j in range(4):
            q_ref[j, :, :] = acc[:, j * _D:(j + 1) * _D]
        sq = acc * acc
        cj = jax.lax.broadcasted_iota(jnp.int32, (4, 512), 0)
        cc = jax.lax.broadcasted_iota(jnp.int32, (4, 512), 1)
        sel = (cc // _D == cj).astype(jnp.float32)
        n2_ref[0] = jax.lax.dot_general(
            sel, sq, (((1,), (1,)), ((), ())), preferred_element_type=jnp.float32)

    @pl.when(s > 0)
    def _():
        for j in range(4):
            kv_ref[0, j, :, :] = acc[:, j * _D:(j + 1) * _D].astype(jnp.bfloat16)


def _qkv_proj(x, w, b):
    return pl.pallas_call(
        _qkv_kernel,
        grid=(4, 3),
        in_specs=[
            pl.BlockSpec((_N, _C), lambda hh, s: (0, 0)),
            pl.BlockSpec((_C, 512), lambda hh, s: (0, s * 4 + hh)),
            pl.BlockSpec((1, 1, 512), lambda hh, s: (s * 4 + hh, 0, 0)),
        ],
        out_specs=[
            pl.BlockSpec((4, _N, _D), lambda hh, s: (hh, 0, 0)),
            pl.BlockSpec((1, 4, _N, _D),
                         lambda hh, s: (jnp.maximum(s - 1, 0), hh, 0, 0)),
            pl.BlockSpec((1, 4, _N), lambda hh, s: (hh, 0, 0)),
        ],
        out_shape=[
            jax.ShapeDtypeStruct((_H, _N, _D), jnp.float32),
            jax.ShapeDtypeStruct((2, _H, _N, _D), jnp.bfloat16),
            jax.ShapeDtypeStruct((4, 4, _N), jnp.float32),
        ],
        scratch_shapes=[pltpu.VMEM((_N, _C), jnp.bfloat16)],
    )(x, w, b)


def _topk_kernel(n2_ref, idx_ref):
    bits = jax.lax.bitcast_convert_type(n2_ref[...], jnp.int32)
    col = jax.lax.broadcasted_iota(jnp.int32, (_H, _N), 1)
    keys = (bits & ~jnp.int32(2047)) | (jnp.int32(2047) - col)
    picks = []
    for _ in range(_U):
        m = jnp.max(keys, axis=1, keepdims=True)
        picks.append(m)
        keys = jnp.where(keys == m, jnp.iinfo(jnp.int32).min, keys)
    mkeys = jnp.concatenate(picks, axis=1)
    idx = jnp.int32(2047) - (mkeys & jnp.int32(2047))
    head = jax.lax.broadcasted_iota(jnp.int32, (_H, _U), 0)
    idx_ref[:, 0, :] = idx + head * _N


def _topk(n2):
    return pl.pallas_call(
        _topk_kernel,
        grid=(1,),
        in_specs=[pl.BlockSpec((_H, _N), lambda i: (0, 0))],
        out_specs=pl.BlockSpec((_H, 1, _U), lambda i: (0, 0, 0)),
        out_shape=jax.ShapeDtypeStruct((_H, 1, _U), jnp.int32),
    )(n2)


def _sc_gather(q_rows, idx_flat):
    mesh = plsc.VectorSubcoreMesh(core_axis_name="c", subcore_axis_name="s")

    @functools.partial(
        pl.kernel, mesh=mesh,
        out_type=jax.ShapeDtypeStruct((_H * _U, _D), jnp.float32),
        scratch_types=[
            pltpu.VMEM((_U,), jnp.int32),
            pltpu.VMEM((_U, _D), jnp.float32),
            pltpu.SemaphoreType.DMA,
        ],
    )
    def gather(table_hbm, idx_hbm, out_hbm, idx_v, rows_v, sem):
        wid = lax.axis_index("s") * 2 + lax.axis_index("c")

        @pl.when(wid < _H)
        def _():
            base = wid * _U
            pltpu.sync_copy(idx_hbm.at[pl.ds(base, _U)], idx_v)
            pltpu.async_copy(table_hbm.at[idx_v], rows_v, sem).wait()
            pltpu.sync_copy(rows_v, out_hbm.at[pl.ds(base, _U)])

    return gather(q_rows, idx_flat)


def _attn_kernel(qred_ref, k_ref, v_ref, idx_ref, out_ref):
    h = pl.program_id(0)
    q_red = qred_ref[0].astype(jnp.bfloat16)
    k = k_ref[0, 0]
    v = v_ref[0, 0]
    s = jax.lax.dot_general(
        q_red, k, (((1,), (1,)), ((), ())),
        preferred_element_type=jnp.float32) * _SCALE

    p = s - jnp.max(s, axis=1, keepdims=True)
    p = jnp.exp(p)
    p = p / jnp.sum(p, axis=1, keepdims=True)
    p2 = p - jnp.max(p, axis=1, keepdims=True)
    p2 = jnp.exp(p2)
    p2 = p2 / jnp.sum(p2, axis=1, keepdims=True)

    rows = jax.lax.dot_general(
        p2.astype(jnp.bfloat16), v, (((1,), (0,)), ((), ())),
        preferred_element_type=jnp.float32)
    rows_aug = jnp.concatenate(
        [rows, jnp.ones((_U, _D), jnp.float32)], axis=1).astype(jnp.bfloat16)

    local_idx = idx_ref[0, 0:1, :] - h * _N
    iota_n = jax.lax.broadcasted_iota(jnp.int32, (_N, _U), 0)
    onehot_t = (iota_n == local_idx).astype(jnp.bfloat16)
    scat = jax.lax.dot_general(
        onehot_t, rows_aug, (((1,), (0,)), ((), ())),
        preferred_element_type=jnp.float32)
    out = jnp.where(scat[:, _D:_D + 1] > 0.0, scat[:, :_D], jnp.nan)
    out_ref[...] = out.astype(jnp.bfloat16)


def _sparse_attn(q_red, kv, idx_flat):
    return pl.pallas_call(
        _attn_kernel,
        grid=(_H,),
        in_specs=[
            pl.BlockSpec((1, _U, _D), lambda h: (h, 0, 0)),
            pl.BlockSpec((1, 1, _N, _D), lambda h: (0, h, 0, 0)),
            pl.BlockSpec((1, 1, _N, _D), lambda h: (1, h, 0, 0)),
            pl.BlockSpec((1, 1, _U), lambda h: (h, 0, 0)),
        ],
        out_specs=pl.BlockSpec((_N, _D), lambda h: (0, h)),
        out_shape=jax.ShapeDtypeStruct((_N, _C), jnp.bfloat16),
    )(q_red, kv, kv, idx_flat)


def _fc_kernel(x_ref, w_ref, b_ref, out_ref, wbf_ref):
    @pl.when(pl.program_id(0) == 0)
    def _():
        wbf_ref[...] = w_ref[...].astype(jnp.bfloat16)

    acc = jax.lax.dot_general(
        x_ref[...], wbf_ref[...], (((1,), (0,)), ((), ())),
        preferred_element_type=jnp.float32)
    out_ref[...] = acc + b_ref[...]


def _fc(attn_out, w, b):
    return pl.pallas_call(
        _fc_kernel,
        grid=(4,),
        in_specs=[
            pl.BlockSpec((512, _C), lambda i: (i, 0)),
            pl.BlockSpec((_C, _C), lambda i: (0, 0)),
            pl.BlockSpec((1, _C), lambda i: (0, 0)),
        ],
        out_specs=pl.BlockSpec((512, _C), lambda i: (i, 0)),
        out_shape=jax.ShapeDtypeStruct((_N, _C), jnp.float32),
        scratch_shapes=[pltpu.VMEM((_C, _C), jnp.bfloat16)],
    )(attn_out, w, b)


@jax.jit
def kernel(query, W_qkv, b_qkv, W_fc, b_fc):
    B, N, C = query.shape
    q, kv, n2 = _qkv_proj(query.reshape(N, C), W_qkv, b_qkv.reshape(12, 1, 512))
    idx_flat = _topk(n2.reshape(_H, _N))
    q_red = _sc_gather(q.reshape(_H * _N, _D), idx_flat.reshape(_H * _U))
    attn_out = _sparse_attn(q_red.reshape(_H, _U, _D), kv, idx_flat)
    out = _fc(attn_out, W_fc, b_fc.reshape(1, C))
    return out.reshape(B, N, C)


# speedup vs baseline: 1.4164x; 1.1069x over previous
import functools

import jax
import jax.numpy as jnp
from jax import lax
from jax.experimental import pallas as pl
from jax.experimental.pallas import tpu as pltpu
from jax.experimental.pallas import tpu_sc as plsc

_N = 2048
_C = 2048
_H = 16
_D = 128
_U = 40
_SCALE = _D ** -0.5


def _qkv_kernel(x_ref, w_ref, b_ref, q_ref, k_ref, v_ref, n2_ref, xbf_ref):
    hh = pl.program_id(0)
    s = pl.program_id(1)

    @pl.when((hh == 0) & (s == 0))
    def _():
        xbf_ref[...] = x_ref[...].astype(jnp.bfloat16)

    acc = jax.lax.dot_general(
        xbf_ref[...], w_ref[...].astype(jnp.bfloat16), (((1,), (0,)), ((), ())),
        preferred_element_type=jnp.float32)
    acc = acc + b_ref[0]

    @pl.when(s == 0)
    def _():
        for j in range(4):
            q_ref[j, :, :] = acc[:, j * _D:(j + 1) * _D]
        sq = acc * acc
        cj = jax.lax.broadcasted_iota(jnp.int32, (4, 512), 0)
        cc = jax.lax.broadcasted_iota(jnp.int32, (4, 512), 1)
        sel = (cc // _D == cj).astype(jnp.float32)
        n2_ref[0] = jax.lax.dot_general(
            sel, sq, (((1,), (1,)), ((), ())), preferred_element_type=jnp.float32)

    @pl.when(s == 1)
    def _():
        for j in range(4):
            k_ref[j, :, :] = acc[:, j * _D:(j + 1) * _D]

    @pl.when(s == 2)
    def _():
        for j in range(4):
            v_ref[j, :, :] = acc[:, j * _D:(j + 1) * _D].astype(jnp.bfloat16)


def _qkv_proj(x, w, b):
    return pl.pallas_call(
        _qkv_kernel,
        grid=(4, 3),
        in_specs=[
            pl.BlockSpec((_N, _C), lambda hh, s: (0, 0)),
            pl.BlockSpec((_C, 512), lambda hh, s: (0, s * 4 + hh)),
            pl.BlockSpec((1, 1, 512), lambda hh, s: (s * 4 + hh, 0, 0)),
        ],
        out_specs=[
            pl.BlockSpec((4, _N, _D), lambda hh, s: (hh, 0, 0)),
            pl.BlockSpec((4, _N, _D), lambda hh, s: (hh, 0, 0)),
            pl.BlockSpec((4, _N, _D), lambda hh, s: (hh, 0, 0)),
            pl.BlockSpec((1, 4, _N), lambda hh, s: (hh, 0, 0)),
        ],
        out_shape=[
            jax.ShapeDtypeStruct((_H, _N, _D), jnp.float32),
            jax.ShapeDtypeStruct((_H, _N, _D), jnp.float32),
            jax.ShapeDtypeStruct((_H, _N, _D), jnp.bfloat16),
            jax.ShapeDtypeStruct((4, 4, _N), jnp.float32),
        ],
        scratch_shapes=[pltpu.VMEM((_N, _C), jnp.bfloat16)],
    )(x, w, b)


def _topk_kernel(n2_ref, idx_ref, idx0_ref):
    bits = jax.lax.bitcast_convert_type(n2_ref[...], jnp.int32)
    col = jax.lax.broadcasted_iota(jnp.int32, (_H, _N), 1)
    keys = (bits & ~jnp.int32(2047)) | (jnp.int32(2047) - col)
    picks = []
    for _ in range(_U):
        m = jnp.max(keys, axis=1, keepdims=True)
        picks.append(m)
        keys = jnp.where(keys == m, jnp.iinfo(jnp.int32).min, keys)
    mkeys = jnp.concatenate(picks, axis=1)
    idx = jnp.int32(2047) - (mkeys & jnp.int32(2047))
    head = jax.lax.broadcasted_iota(jnp.int32, (_H, _U), 0)
    idx_ref[:, 0, :] = idx + head * _N
    e0 = (jax.lax.broadcasted_iota(jnp.int32, (_H, 1), 0) == 0)
    idx0_ref[...] = jax.lax.dot_general(
        idx.astype(jnp.float32), e0.astype(jnp.float32),
        (((0,), (0,)), ((), ())), preferred_element_type=jnp.float32)


def _topk(n2):
    return pl.pallas_call(
        _topk_kernel,
        grid=(1,),
        in_specs=[pl.BlockSpec((_H, _N), lambda i: (0, 0))],
        out_specs=[
            pl.BlockSpec((_H, 1, _U), lambda i: (0, 0, 0)),
            pl.BlockSpec((_U, 1), lambda i: (0, 0)),
        ],
        out_shape=[
            jax.ShapeDtypeStruct((_H, 1, _U), jnp.int32),
            jax.ShapeDtypeStruct((_U, 1), jnp.float32),
        ],
    )(n2)


def _sc_gather(q_rows, idx_flat):
    mesh = plsc.VectorSubcoreMesh(core_axis_name="c", subcore_axis_name="s")

    @functools.partial(
        pl.kernel, mesh=mesh,
        out_type=jax.ShapeDtypeStruct((_H * _U, _D), jnp.float32),
        scratch_types=[
            pltpu.VMEM((_U,), jnp.int32),
            pltpu.VMEM((_U, _D), jnp.float32),
            pltpu.SemaphoreType.DMA,
        ],
    )
    def gather(table_hbm, idx_hbm, out_hbm, idx_v, rows_v, sem):
        wid = lax.axis_index("s") * 2 + lax.axis_index("c")

        @pl.when(wid < _H)
        def _():
            base = wid * _U
            pltpu.sync_copy(idx_hbm.at[pl.ds(base, _U)], idx_v)
            pltpu.async_copy(table_hbm.at[idx_v], rows_v, sem).wait()
            pltpu.sync_copy(rows_v, out_hbm.at[pl.ds(base, _U)])

    return gather(q_rows, idx_flat)


def _attn_kernel(qred_ref, k_ref, v_ref, idx_ref, idx0_ref, out_ref):
    h = pl.program_id(0)
    q_red = qred_ref[0]
    k = k_ref[0]
    v = v_ref[0]
    s = jax.lax.dot_general(
        q_red, k, (((1,), (1,)), ((), ())),
        preferred_element_type=jnp.float32) * _SCALE

    p = s - jnp.max(s, axis=1, keepdims=True)
    p = jnp.exp(p)
    p = p / jnp.sum(p, axis=1, keepdims=True)
    p2 = p - jnp.max(p, axis=1, keepdims=True)
    p2 = jnp.exp(p2)
    p2 = p2 / jnp.sum(p2, axis=1, keepdims=True)

    rows = jax.lax.dot_general(
        p2.astype(jnp.bfloat16), v, (((1,), (0,)), ((), ())),
        preferred_element_type=jnp.float32)
    rows_aug = jnp.concatenate(
        [rows, jnp.ones((_U, _D), jnp.float32)], axis=1)

    local_idx = (idx_ref[0, 0:1, :] - h * _N).astype(jnp.float32)
    match = (idx0_ref[...] == local_idx).astype(jnp.float32)
    out_ref[0] = jax.lax.dot_general(
        match, rows_aug, (((1,), (0,)), ((), ())),
        preferred_element_type=jnp.float32)


def _sparse_attn(q_red, k, v, idx_flat, idx0_col):
    return pl.pallas_call(
        _attn_kernel,
        grid=(_H,),
        in_specs=[
            pl.BlockSpec((1, _U, _D), lambda h: (h, 0, 0)),
            pl.BlockSpec((1, _N, _D), lambda h: (h, 0, 0)),
            pl.BlockSpec((1, _N, _D), lambda h: (h, 0, 0)),
            pl.BlockSpec((1, 1, _U), lambda h: (h, 0, 0)),
            pl.BlockSpec((_U, 1), lambda h: (0, 0)),
        ],
        out_specs=pl.BlockSpec((1, _U, 2 * _D), lambda h: (h, 0, 0)),
        out_shape=jax.ShapeDtypeStruct((_H, _U, 2 * _D), jnp.float32),
    )(q_red, k, v, idx_flat, idx0_col)


def _fc_kernel(cand_ref, idx_ref, w_ref, b_ref, out_ref, aug_ref):
    i = pl.program_id(0)

    @pl.when(i == 0)
    def _():
        cand = jnp.concatenate(
            [cand_ref[h, :, :_D] for h in range(_H)], axis=1)
        inter = cand_ref[0, :, _D:_D + 1]
        for h in range(1, _H):
            inter = jnp.minimum(inter, cand_ref[h, :, _D:_D + 1])
        fc = jax.lax.dot_general(
            cand, w_ref[...], (((1,), (0,)), ((), ())),
            preferred_element_type=jnp.float32) + b_ref[...]
        aug_ref[...] = jnp.concatenate(
            [fc, inter * jnp.ones((_U, _D), jnp.float32)], axis=1)

    idx0 = idx_ref[0, 0:1, :].astype(jnp.float32)
    row = (jax.lax.broadcasted_iota(jnp.int32, (512, _U), 0)
           + i * 512).astype(jnp.float32)
    onehot = (row == idx0).astype(jnp.float32)
    scat = jax.lax.dot_general(
        onehot, aug_ref[...], (((1,), (0,)), ((), ())),
        preferred_element_type=jnp.float32)
    out_ref[...] = jnp.where(scat[:, _C:_C + 1] > 0.5, scat[:, :_C], jnp.nan)


def _fc(cand, idx_flat, w, b):
    return pl.pallas_call(
        _fc_kernel,
        grid=(4,),
        in_specs=[
            pl.BlockSpec((_H, _U, 2 * _D), lambda i: (0, 0, 0)),
            pl.BlockSpec((1, 1, _U), lambda i: (0, 0, 0)),
            pl.BlockSpec((_C, _C), lambda i: (0, 0)),
            pl.BlockSpec((1, _C), lambda i: (0, 0)),
        ],
        out_specs=pl.BlockSpec((512, _C), lambda i: (i, 0)),
        out_shape=jax.ShapeDtypeStruct((_N, _C), jnp.float32),
        scratch_shapes=[pltpu.VMEM((_U, _C + _D), jnp.float32)],
    )(cand, idx_flat, w, b)


@jax.jit
def kernel(query, W_qkv, b_qkv, W_fc, b_fc):
    B, N, C = query.shape
    q, k, v, n2 = _qkv_proj(query.reshape(N, C), W_qkv, b_qkv.reshape(12, 1, 512))
    idx_flat, idx0_col = _topk(n2.reshape(_H, _N))
    q_red = _sc_gather(q.reshape(_H * _N, _D), idx_flat.reshape(_H * _U))
    cand = _sparse_attn(q_red.reshape(_H, _U, _D), k, v, idx_flat, idx0_col)
    out = _fc(cand, idx_flat, W_fc, b_fc.reshape(1, C))
    return out.reshape(B, N, C)
